# baseline (speedup 1.0000x reference)
import functools

import jax
import jax.numpy as jnp
from jax import lax
from jax.experimental import pallas as pl
from jax.experimental.pallas import tpu as pltpu
from jax.experimental.pallas import tpu_sc as plsc

_NCORES = 2
_NSUB = 16
_NW = _NCORES * _NSUB
_LANES = 16
_NBINS = 4096
_BIN_SHIFT = 20
_NPRIV = 8
_NV = 16
_CCAP = (_NV + 1) * _LANES


def _monotonic_u32(xv):
  bits = lax.bitcast_convert_type(xv, jnp.int32)
  flip = (bits >> 31) | jnp.int32(-2**31)
  u = lax.bitcast_convert_type(bits ^ flip, jnp.uint32)
  return jnp.where(xv == 0.0, jnp.uint32(0x80000000), u)


def _topk_body(n, k, rows_per_w, scores_hbm, out_idx_hbm, out_val_hbm,
               x_v, cidx_v, hist_v, oval_v, oidx_v, sem_v):
  nchunks = n // _LANES
  wid = lax.axis_index("s") * _NCORES + lax.axis_index("c")
  lane = lax.iota(jnp.int32, _LANES)
  neg_inf = jnp.float32(-jnp.inf)
  big = jnp.int32(2**30)
  lane15 = jnp.full((_LANES,), 15, jnp.int32)
  lane7 = lane & 7
  lanehi = lane >= 8
  lane0 = lane == 0
  zeros16 = jnp.zeros((_LANES,), jnp.int32)
  ninf16 = jnp.full((_LANES,), neg_inf, jnp.float32)

  def _perm(v, idx):
    return jnp.take_along_axis(v, idx, axis=0)

  def _combine(av, ai, bv, bi):
    better = (bv > av) | ((bv == av) & (bi < ai))
    return jnp.where(better, bv, av), jnp.where(better, bi, ai)

  def _argmax_splat(bv, bi):
    for sh in (8, 4, 2, 1):
      pidx = lane ^ sh
      bv, bi = _combine(bv, bi, _perm(bv, pidx), _perm(bi, pidx))
    return bv, bi

  nslot = n + 128
  x_v[pl.ds(n, _LANES)] = ninf16
  x_v[pl.ds(nslot + n, _LANES)] = ninf16

  def _copy_slot(r, slot_base):
    return pltpu.make_async_copy(
        scores_hbm.at[wid * rows_per_w + r],
        x_v.at[pl.ds(slot_base, n)],
        sem_v.at[(slot_base > 0) * 1])

  def _start(r):
    @pl.when((r & 1) == 0)
    def _():
      _copy_slot(r, 0).start()

    @pl.when((r & 1) == 1)
    def _():
      _copy_slot(r, nslot).start()

  def _wait(r):
    @pl.when((r & 1) == 0)
    def _():
      _copy_slot(r, 0).wait()

    @pl.when((r & 1) == 1)
    def _():
      _copy_slot(r, nslot).wait()

  @pl.when(rows_per_w > 0)
  def _():
    _start(jnp.int32(0))

  def do_row(r, _):
    row = wid * rows_per_w + r
    base = (r & 1) * nslot
    _wait(r)

    @pl.when(r + 1 < rows_per_w)
    def _():
      _start(r + 1)

    @plsc.parallel_loop(0, _NBINS * _NPRIV // _LANES, unroll=8)
    def _(i):
      hist_v[pl.ds(i * _LANES, _LANES)] = zeros16

    ones = jnp.ones((_LANES,), jnp.int32)

    @plsc.parallel_loop(0, nchunks, unroll=8)
    def _(i):
      xv = x_v[pl.ds(base + i * _LANES, _LANES)]
      u = _monotonic_u32(xv)
      d = (u >> _BIN_SHIFT).astype(jnp.int32)
      plsc.addupdate_scatter(hist_v, [(d << 3) | lane7, ], ones, mask=lane >= 0)

    def scond(c):
      b, cum = c
      return (cum < k) & (b >= 0)

    def sbody(c):
      b, cum = c
      hv = hist_v[pl.ds(b * _LANES, _LANES)]
      return b - 1, cum + jnp.sum(hv)
    cfin, cum = lax.while_loop(
        scond, sbody,
        (jnp.int32(_NBINS * _NPRIV // _LANES - 1), jnp.int32(0)))
    cc = jnp.maximum(cfin + 1, 0)
    hv = hist_v[pl.ds(cc * _LANES, _LANES)]
    s_all = jnp.sum(hv)
    s_hi = jnp.sum(jnp.where(lanehi, hv, 0))
    cum_above = cum - s_all
    binsel = jnp.where(cum >= k,
                       2 * cc + jnp.where(cum_above + s_hi >= k, 1, 0),
                       0)
    thresh = binsel.astype(jnp.uint32) << _BIN_SHIFT

    @plsc.parallel_loop(0, _CCAP // _LANES, unroll=4)
    def _(j):
      cidx_v[pl.ds(j * _LANES, _LANES)] = lane + jnp.int32(n)

    @plsc.parallel_loop(0, nchunks, unroll=4, carry=zeros16)
    def cntv(i, cntv):
      xv = x_v[pl.ds(base + i * _LANES, _LANES)]
      u = _monotonic_u32(xv)
      m = u >= thresh
      iv = lane + i * _LANES
      cs = plsc.cumsum(m.astype(jnp.int32))
      pos = cntv + cs - 1
      plsc.store_scatter(cidx_v, [pos], iv, mask=m & (pos < _CCAP))
      return cntv + _perm(cs, lane15)
    cnt = cntv[0]
    base16 = zeros16 + base

    def fast_sel():
      ivs = [cidx_v[pl.ds(j * _LANES, _LANES)] for j in range(_NV)]
      vvs0 = tuple(plsc.load_gather(x_v, [iv + base16]) for iv in ivs)

      def step(j, vvs):
        red = [(vvs[t], ivs[t]) for t in range(_NV)]
        while len(red) > 1:
          half = len(red) // 2
          red = [( _combine(*red[t], *red[t + half]) ) for t in range(half)]
        bv, bi = _argmax_splat(*red[0])
        jsplat = zeros16 + j
        plsc.store_scatter(oval_v, [jsplat], bv, mask=lane0)
        plsc.store_scatter(oidx_v, [jsplat], bi, mask=lane0)
        return tuple(jnp.where(ivs[t] == bi, neg_inf, vvs[t])
                     for t in range(_NV))
      lax.fori_loop(0, k, step, vvs0)

    def slow_sel():
      init_sel = (ninf16, jnp.full((_LANES,), big, jnp.int32))

      def sel(j, _):
        @plsc.parallel_loop(0, nchunks, unroll=2, carry=init_sel)
        def sweep(v, carry):
          bv, bi = carry
          iv = lane + v * _LANES
          vv = x_v[pl.ds(base + v * _LANES, _LANES)]
          return _combine(bv, bi, vv, iv)
        bv, bi = _argmax_splat(*sweep)
        jsplat = zeros16 + j
        plsc.store_scatter(oval_v, [jsplat], bv, mask=lane0)
        plsc.store_scatter(oidx_v, [jsplat], bi, mask=lane0)
        plsc.store_scatter(x_v, [bi + base16], ninf16, mask=lane0)
        return 0
      lax.fori_loop(0, k, sel, 0)

    lax.cond(cnt <= _NV * _LANES, fast_sel, slow_sel)

    pltpu.sync_copy(oidx_v, out_idx_hbm.at[row])
    pltpu.sync_copy(oval_v, out_val_hbm.at[row])
    return 0

  lax.fori_loop(0, rows_per_w, do_row, 0)


def kernel(scores):
  b, n = scores.shape
  k = min(64, n)
  assert b % _NW == 0 and n % _LANES == 0
  rows_per_w = b // _NW
  body = functools.partial(_topk_body, n, k, rows_per_w)
  f = pl.kernel(
      body,
      out_type=(jax.ShapeDtypeStruct((b, k), jnp.int32),
                jax.ShapeDtypeStruct((b, k), jnp.float32)),
      mesh=plsc.VectorSubcoreMesh(core_axis_name="c", subcore_axis_name="s",
                                  num_cores=_NCORES, num_subcores=_NSUB),
      compiler_params=pltpu.CompilerParams(needs_layout_passes=False),
      scratch_types=[
          pltpu.VMEM((2 * (n + 128),), jnp.float32),
          pltpu.VMEM(((_NV + 1) * _LANES,), jnp.int32),
          pltpu.VMEM((_NBINS * _NPRIV,), jnp.int32),
          pltpu.VMEM((64,), jnp.float32),
          pltpu.VMEM((64,), jnp.int32),
          pltpu.SemaphoreType.DMA((2,)),
      ],
  )
  return f(scores)


# speedup vs baseline: 1.7838x; 1.7838x over previous
import functools

import jax
import jax.numpy as jnp
from jax import lax
from jax.experimental import pallas as pl
from jax.experimental.pallas import tpu as pltpu
from jax.experimental.pallas import tpu_sc as plsc

_NCORES = 2
_NSUB = 16
_NW = _NCORES * _NSUB
_LANES = 16
_NBINS = 4096
_BIN_SHIFT = 20
_NV = 16


def _monotonic_u32(xv):
  bits = lax.bitcast_convert_type(xv, jnp.int32)
  flip = (bits >> 31) | jnp.int32(-2**31)
  u = lax.bitcast_convert_type(bits ^ flip, jnp.uint32)
  return jnp.where(xv == 0.0, jnp.uint32(0x80000000), u)


def _topk_body(n, k, rows_per_w, scores_hbm, out_idx_hbm, out_val_hbm,
               x_v, cidx_v, hist_v, oval_v, oidx_v, sem_v, osem_v):
  nchunks = n // _LANES
  nslot = n + 128
  wid = lax.axis_index("s") * _NCORES + lax.axis_index("c")
  lane = lax.iota(jnp.int32, _LANES)
  neg_inf = jnp.float32(-jnp.inf)
  big = jnp.int32(2**30)
  lane15 = jnp.full((_LANES,), 15, jnp.int32)
  lane0 = lane == 0
  zeros16 = jnp.zeros((_LANES,), jnp.int32)
  ninf16 = jnp.full((_LANES,), neg_inf, jnp.float32)

  def _perm(v, idx):
    return jnp.take_along_axis(v, idx, axis=0)

  def _combine(av, ai, bv, bi):
    better = (bv > av) | ((bv == av) & (bi < ai))
    return jnp.where(better, bv, av), jnp.where(better, bi, ai)

  def _argmax_splat(bv, bi):
    for sh in (8, 4, 2, 1):
      pidx = lane ^ sh
      bv, bi = _combine(bv, bi, _perm(bv, pidx), _perm(bi, pidx))
    return bv, bi

  x_v[pl.ds(n, _LANES)] = ninf16
  x_v[pl.ds(nslot + n, _LANES)] = ninf16

  def _copy_slot(r, slot_base):
    return pltpu.make_async_copy(
        scores_hbm.at[wid * rows_per_w + r],
        x_v.at[pl.ds(slot_base, n)],
        sem_v.at[(slot_base > 0) * 1])

  def _start(r):
    @pl.when((r & 1) == 0)
    def _():
      _copy_slot(r, 0).start()

    @pl.when((r & 1) == 1)
    def _():
      _copy_slot(r, nslot).start()

  def _wait(r):
    @pl.when((r & 1) == 0)
    def _():
      _copy_slot(r, 0).wait()

    @pl.when((r & 1) == 1)
    def _():
      _copy_slot(r, nslot).wait()

  @pl.when(rows_per_w > 0)
  def _():
    _start(jnp.int32(0))

  def do_row(r, _):
    base = (r & 1) * nslot
    _wait(r)

    @pl.when(r + 1 < rows_per_w)
    def _():
      _start(r + 1)

    @plsc.parallel_loop(0, _NBINS // _LANES, unroll=8)
    def _(i):
      hist_v[pl.ds(i * _LANES, _LANES)] = zeros16

    ones = jnp.ones((_LANES,), jnp.int32)

    @plsc.parallel_loop(0, nchunks, unroll=8)
    def _(i):
      xv = x_v[pl.ds(base + i * _LANES, _LANES)]
      u = _monotonic_u32(xv)
      d = (u >> _BIN_SHIFT).astype(jnp.int32)
      plsc.addupdate_scatter(hist_v, [d], ones, mask=lane >= 0)

    def scond(c):
      b, cum = c
      return (cum < k) & (b >= 0)

    def sbody(c):
      b, cum = c
      hv = hist_v[pl.ds(b * _LANES, _LANES)]
      return b - 1, cum + jnp.sum(hv)
    cfin, cum = lax.while_loop(scond, sbody,
                               (jnp.int32(_NBINS // _LANES - 1), jnp.int32(0)))
    cc = jnp.maximum(cfin + 1, 0)
    hv = hist_v[pl.ds(cc * _LANES, _LANES)]
    cum_above = cum - jnp.sum(hv)
    cs = jnp.cumsum(lax.rev(hv, (0,)))
    p = jnp.sum((cum_above + cs < k).astype(jnp.int32))
    binsel = jnp.where(cum >= k, cc * _LANES + (_LANES - 1) - p, 0)
    thresh = binsel.astype(jnp.uint32) << _BIN_SHIFT

    @plsc.parallel_loop(0, _NV, unroll=4)
    def _(j):
      cidx_v[pl.ds(j * _LANES, _LANES)] = lane + jnp.int32(n)

    @plsc.parallel_loop(0, nchunks, unroll=8, carry=zeros16)
    def cntv(i, cntv):
      xv = x_v[pl.ds(base + i * _LANES, _LANES)]
      u = _monotonic_u32(xv)
      m = u >= thresh
      iv = lane + i * _LANES
      cs = plsc.cumsum(m.astype(jnp.int32))
      plsc.store_scatter(cidx_v, [cntv + cs - 1], iv, mask=m)
      return cntv + _perm(cs, lane15)
    cnt = cntv[0]
    base16 = zeros16 + base
    rsplat = zeros16 + r

    def make_fast(nv):
      def fast_sel():
        ivs = [cidx_v[pl.ds(j * _LANES, _LANES)] for j in range(nv)]
        vvs0 = tuple(plsc.load_gather(x_v, [iv + base16]) for iv in ivs)

        def step(j, vvs):
          red = [(vvs[t], ivs[t]) for t in range(nv)]
          while len(red) > 1:
            half = len(red) // 2
            red = [_combine(*red[t], *red[t + half]) for t in range(half)]
          bv, bi = _argmax_splat(*red[0])
          jsplat = zeros16 + j
          plsc.store_scatter(oval_v, [rsplat, jsplat], bv, mask=lane0)
          plsc.store_scatter(oidx_v, [rsplat, jsplat], bi, mask=lane0)
          return tuple(jnp.where(ivs[t] == bi, neg_inf, vvs[t])
                       for t in range(nv))
        lax.fori_loop(0, k, step, vvs0)
      return fast_sel

    def slow_sel():
      cidx_v[pl.ds(cnt, _LANES)] = lane + jnp.int32(n)
      nvec = (cnt + _LANES - 1) // _LANES
      init_sel = (ninf16, jnp.full((_LANES,), big, jnp.int32))

      def sel(j, _):
        @plsc.parallel_loop(0, nvec, unroll=2, carry=init_sel)
        def sweep(v, carry):
          bv, bi = carry
          iv = cidx_v[pl.ds(v * _LANES, _LANES)]
          vv = plsc.load_gather(x_v, [iv + base16])
          return _combine(bv, bi, vv, iv)
        bv, bi = _argmax_splat(*sweep)
        jsplat = zeros16 + j
        plsc.store_scatter(oval_v, [rsplat, jsplat], bv, mask=lane0)
        plsc.store_scatter(oidx_v, [rsplat, jsplat], bi, mask=lane0)
        plsc.store_scatter(x_v, [bi + base16], ninf16, mask=lane0)
        return 0
      lax.fori_loop(0, k, sel, 0)

    lax.cond(cnt <= 8 * _LANES, make_fast(8),
             lambda: lax.cond(cnt <= _NV * _LANES, make_fast(_NV), slow_sel))
    return 0

  lax.fori_loop(0, rows_per_w, do_row, 0)

  c0 = pltpu.make_async_copy(
      oidx_v, out_idx_hbm.at[pl.ds(wid * rows_per_w, rows_per_w)],
      osem_v.at[0])
  c1 = pltpu.make_async_copy(
      oval_v, out_val_hbm.at[pl.ds(wid * rows_per_w, rows_per_w)],
      osem_v.at[1])
  c0.start()
  c1.start()
  c0.wait()
  c1.wait()


def kernel(scores):
  b, n = scores.shape
  k = min(64, n)
  assert b % _NW == 0 and n % _LANES == 0
  rows_per_w = b // _NW
  body = functools.partial(_topk_body, n, k, rows_per_w)
  f = pl.kernel(
      body,
      out_type=(jax.ShapeDtypeStruct((b, k), jnp.int32),
                jax.ShapeDtypeStruct((b, k), jnp.float32)),
      mesh=plsc.VectorSubcoreMesh(core_axis_name="c", subcore_axis_name="s",
                                  num_cores=_NCORES, num_subcores=_NSUB),
      compiler_params=pltpu.CompilerParams(needs_layout_passes=False),
      scratch_types=[
          pltpu.VMEM((2 * (n + 128),), jnp.float32),
          pltpu.VMEM((n + _LANES,), jnp.int32),
          pltpu.VMEM((_NBINS,), jnp.int32),
          pltpu.VMEM((b // _NW, 64), jnp.float32),
          pltpu.VMEM((b // _NW, 64), jnp.int32),
          pltpu.SemaphoreType.DMA((2,)),
          pltpu.SemaphoreType.DMA((2,)),
      ],
  )
  return f(scores)


# speedup vs baseline: 2.2561x; 1.2648x over previous
import functools

import jax
import jax.numpy as jnp
from jax import lax
from jax.experimental import pallas as pl
from jax.experimental.pallas import tpu as pltpu
from jax.experimental.pallas import tpu_sc as plsc

_NCORES = 2
_NSUB = 16
_NW = _NCORES * _NSUB
_LANES = 16
_NBINS = 4096
_BIN_SHIFT = 20
_NV = 16


def _monotonic_u32(xv):
  bits = lax.bitcast_convert_type(xv, jnp.int32)
  flip = (bits >> 31) | jnp.int32(-2**31)
  u = lax.bitcast_convert_type(bits ^ flip, jnp.uint32)
  return jnp.where(xv == 0.0, jnp.uint32(0x80000000), u)


def _topk_body(n, k, rows_per_w, scores_hbm, out_idx_hbm, out_val_hbm,
               x_v, cidx_v, hist_v, oval_v, oidx_v, sem_v, osem_v):
  nchunks = n // _LANES
  nslot = n + 128
  wid = lax.axis_index("s") * _NCORES + lax.axis_index("c")
  lane = lax.iota(jnp.int32, _LANES)
  neg_inf = jnp.float32(-jnp.inf)
  big = jnp.int32(2**30)
  lane15 = jnp.full((_LANES,), 15, jnp.int32)
  lane0 = lane == 0
  zeros16 = jnp.zeros((_LANES,), jnp.int32)
  ninf16 = jnp.full((_LANES,), neg_inf, jnp.float32)

  def _perm(v, idx):
    return jnp.take_along_axis(v, idx, axis=0)

  def _combine(av, ai, bv, bi):
    better = (bv > av) | ((bv == av) & (bi < ai))
    return jnp.where(better, bv, av), jnp.where(better, bi, ai)

  def _argmax_splat(bv, bi):
    for sh in (8, 4, 2, 1):
      pidx = lane ^ sh
      bv, bi = _combine(bv, bi, _perm(bv, pidx), _perm(bi, pidx))
    return bv, bi

  x_v[pl.ds(n, _LANES)] = ninf16
  x_v[pl.ds(nslot + n, _LANES)] = ninf16

  def _copy_slot(r, slot_base):
    return pltpu.make_async_copy(
        scores_hbm.at[wid * rows_per_w + r],
        x_v.at[pl.ds(slot_base, n)],
        sem_v.at[(slot_base > 0) * 1])

  def _start(r):
    @pl.when((r & 1) == 0)
    def _():
      _copy_slot(r, 0).start()

    @pl.when((r & 1) == 1)
    def _():
      _copy_slot(r, nslot).start()

  def _wait(r):
    @pl.when((r & 1) == 0)
    def _():
      _copy_slot(r, 0).wait()

    @pl.when((r & 1) == 1)
    def _():
      _copy_slot(r, nslot).wait()

  @pl.when(rows_per_w > 0)
  def _():
    _start(jnp.int32(0))

  def do_row(r, _):
    base = (r & 1) * nslot
    _wait(r)

    @pl.when(r + 1 < rows_per_w)
    def _():
      _start(r + 1)

    base16 = zeros16 + base
    rsplat = zeros16 + r

    @plsc.parallel_loop(0, _NV, unroll=4)
    def _(j):
      cidx_v[pl.ds(j * _LANES, _LANES)] = lane + jnp.int32(n)

    def compact(thresh):
      @plsc.parallel_loop(0, nchunks, unroll=8, carry=zeros16)
      def cntv(i, cntv):
        xv = x_v[pl.ds(base + i * _LANES, _LANES)]
        u = _monotonic_u32(xv)
        m = u >= thresh
        iv = lane + i * _LANES
        cs = plsc.cumsum(m.astype(jnp.int32))
        plsc.store_scatter(cidx_v, [cntv + cs - 1], iv, mask=m)
        return cntv + _perm(cs, lane15)
      return cntv[0]

    cnt0 = compact(_monotonic_u32(jnp.full((_LANES,), 2.7, jnp.float32))[0])

    def fallback():
      @plsc.parallel_loop(0, _NBINS // _LANES, unroll=8)
      def _(i):
        hist_v[pl.ds(i * _LANES, _LANES)] = zeros16

      ones = jnp.ones((_LANES,), jnp.int32)

      @plsc.parallel_loop(0, nchunks, unroll=8)
      def _(i):
        xv = x_v[pl.ds(base + i * _LANES, _LANES)]
        u = _monotonic_u32(xv)
        d = (u >> _BIN_SHIFT).astype(jnp.int32)
        plsc.addupdate_scatter(hist_v, [d], ones, mask=lane >= 0)

      def scond(c):
        b, cum = c
        return (cum < k) & (b >= 0)

      def sbody(c):
        b, cum = c
        hv = hist_v[pl.ds(b * _LANES, _LANES)]
        return b - 1, cum + jnp.sum(hv)
      cfin, cum = lax.while_loop(
          scond, sbody, (jnp.int32(_NBINS // _LANES - 1), jnp.int32(0)))
      cc = jnp.maximum(cfin + 1, 0)
      hv = hist_v[pl.ds(cc * _LANES, _LANES)]
      cum_above = cum - jnp.sum(hv)
      cs = jnp.cumsum(lax.rev(hv, (0,)))
      p = jnp.sum((cum_above + cs < k).astype(jnp.int32))
      binsel = jnp.where(cum >= k, cc * _LANES + (_LANES - 1) - p, 0)
      thresh = binsel.astype(jnp.uint32) << _BIN_SHIFT

      @plsc.parallel_loop(0, _NV, unroll=4)
      def _(j):
        cidx_v[pl.ds(j * _LANES, _LANES)] = lane + jnp.int32(n)

      tiered_select(compact(thresh))
    def make_fast(nv):
      def fast_sel():
        ivs = [cidx_v[pl.ds(j * _LANES, _LANES)] for j in range(nv)]
        vvs0 = tuple(plsc.load_gather(x_v, [iv + base16]) for iv in ivs)

        def step(j, vvs):
          red = [(vvs[t], ivs[t]) for t in range(nv)]
          while len(red) > 1:
            half = len(red) // 2
            red = [_combine(*red[t], *red[t + half]) for t in range(half)]
          bv, bi = _argmax_splat(*red[0])
          jsplat = zeros16 + j
          plsc.store_scatter(oval_v, [rsplat, jsplat], bv, mask=lane0)
          plsc.store_scatter(oidx_v, [rsplat, jsplat], bi, mask=lane0)
          return tuple(jnp.where(ivs[t] == bi, neg_inf, vvs[t])
                       for t in range(nv))
        lax.fori_loop(0, k, step, vvs0)
      return fast_sel

    def make_slow(cnt):
     def slow_sel():
      cidx_v[pl.ds(cnt, _LANES)] = lane + jnp.int32(n)
      nvec = (cnt + _LANES - 1) // _LANES
      init_sel = (ninf16, jnp.full((_LANES,), big, jnp.int32))

      def sel(j, _):
        @plsc.parallel_loop(0, nvec, unroll=2, carry=init_sel)
        def sweep(v, carry):
          bv, bi = carry
          iv = cidx_v[pl.ds(v * _LANES, _LANES)]
          vv = plsc.load_gather(x_v, [iv + base16])
          return _combine(bv, bi, vv, iv)
        bv, bi = _argmax_splat(*sweep)
        jsplat = zeros16 + j
        plsc.store_scatter(oval_v, [rsplat, jsplat], bv, mask=lane0)
        plsc.store_scatter(oidx_v, [rsplat, jsplat], bi, mask=lane0)
        plsc.store_scatter(x_v, [bi + base16], ninf16, mask=lane0)
        return 0
      lax.fori_loop(0, k, sel, 0)
     return slow_sel

    def tiered_select(cnt):
      lax.cond(cnt <= 8 * _LANES, make_fast(8),
               lambda: lax.cond(cnt <= _NV * _LANES, make_fast(_NV),
                                make_slow(cnt)))

    lax.cond((cnt0 >= k) & (cnt0 <= _NV * _LANES),
             lambda: tiered_select(cnt0), fallback)
    return 0

  lax.fori_loop(0, rows_per_w, do_row, 0)

  c0 = pltpu.make_async_copy(
      oidx_v, out_idx_hbm.at[pl.ds(wid * rows_per_w, rows_per_w)],
      osem_v.at[0])
  c1 = pltpu.make_async_copy(
      oval_v, out_val_hbm.at[pl.ds(wid * rows_per_w, rows_per_w)],
      osem_v.at[1])
  c0.start()
  c1.start()
  c0.wait()
  c1.wait()


def kernel(scores):
  b, n = scores.shape
  k = min(64, n)
  assert b % _NW == 0 and n % _LANES == 0
  rows_per_w = b // _NW
  body = functools.partial(_topk_body, n, k, rows_per_w)
  f = pl.kernel(
      body,
      out_type=(jax.ShapeDtypeStruct((b, k), jnp.int32),
                jax.ShapeDtypeStruct((b, k), jnp.float32)),
      mesh=plsc.VectorSubcoreMesh(core_axis_name="c", subcore_axis_name="s",
                                  num_cores=_NCORES, num_subcores=_NSUB),
      compiler_params=pltpu.CompilerParams(needs_layout_passes=False),
      scratch_types=[
          pltpu.VMEM((2 * (n + 128),), jnp.float32),
          pltpu.VMEM((n + _LANES,), jnp.int32),
          pltpu.VMEM((_NBINS,), jnp.int32),
          pltpu.VMEM((b // _NW, 64), jnp.float32),
          pltpu.VMEM((b // _NW, 64), jnp.int32),
          pltpu.SemaphoreType.DMA((2,)),
          pltpu.SemaphoreType.DMA((2,)),
      ],
  )
  return f(scores)


# speedup vs baseline: 2.6534x; 1.1761x over previous
import functools

import jax
import jax.numpy as jnp
from jax import lax
from jax.experimental import pallas as pl
from jax.experimental.pallas import tpu as pltpu
from jax.experimental.pallas import tpu_sc as plsc

_NCORES = 2
_NSUB = 16
_NW = _NCORES * _NSUB
_LANES = 16
_NBINS = 4096
_BIN_SHIFT = 20
_NV = 16


def _monotonic_u32(xv):
  bits = lax.bitcast_convert_type(xv, jnp.int32)
  flip = (bits >> 31) | jnp.int32(-2**31)
  u = lax.bitcast_convert_type(bits ^ flip, jnp.uint32)
  return jnp.where(xv == 0.0, jnp.uint32(0x80000000), u)


def _topk_body(n, k, rows_per_w, scores_hbm, out_idx_hbm, out_val_hbm,
               x_v, cidx_v, hist_v, oval_v, oidx_v, sem_v, osem_v):
  nchunks = n // _LANES
  nslot = n + 128
  wid = lax.axis_index("s") * _NCORES + lax.axis_index("c")
  lane = lax.iota(jnp.int32, _LANES)
  neg_inf = jnp.float32(-jnp.inf)
  big = jnp.int32(2**30)
  lane15 = jnp.full((_LANES,), 15, jnp.int32)
  lane0 = lane == 0
  zeros16 = jnp.zeros((_LANES,), jnp.int32)
  ninf16 = jnp.full((_LANES,), neg_inf, jnp.float32)

  def _perm(v, idx):
    return jnp.take_along_axis(v, idx, axis=0)

  def _combine(av, ai, bv, bi):
    better = (bv > av) | ((bv == av) & (bi < ai))
    return jnp.where(better, bv, av), jnp.where(better, bi, ai)

  def _argmax_splat(bv, bi):
    for sh in (8, 4, 2, 1):
      pidx = lane ^ sh
      bv, bi = _combine(bv, bi, _perm(bv, pidx), _perm(bi, pidx))
    return bv, bi

  x_v[pl.ds(n, _LANES)] = ninf16
  x_v[pl.ds(nslot + n, _LANES)] = ninf16

  def _copy_slot(r, slot_base):
    return pltpu.make_async_copy(
        scores_hbm.at[wid * rows_per_w + r],
        x_v.at[pl.ds(slot_base, n)],
        sem_v.at[(slot_base > 0) * 1])

  def _start(r):
    @pl.when((r & 1) == 0)
    def _():
      _copy_slot(r, 0).start()

    @pl.when((r & 1) == 1)
    def _():
      _copy_slot(r, nslot).start()

  def _wait(r):
    @pl.when((r & 1) == 0)
    def _():
      _copy_slot(r, 0).wait()

    @pl.when((r & 1) == 1)
    def _():
      _copy_slot(r, nslot).wait()

  @pl.when(rows_per_w > 0)
  def _():
    _start(jnp.int32(0))

  def do_row(r, _):
    base = (r & 1) * nslot
    _wait(r)

    @pl.when(r + 1 < rows_per_w)
    def _():
      _start(r + 1)

    base16 = zeros16 + base
    rsplat = zeros16 + r

    @plsc.parallel_loop(0, _NV, unroll=4)
    def _(j):
      cidx_v[pl.ds(j * _LANES, _LANES)] = lane + jnp.int32(n)

    def compact(thresh):
      @plsc.parallel_loop(0, nchunks, unroll=8, carry=zeros16)
      def cntv(i, cntv):
        xv = x_v[pl.ds(base + i * _LANES, _LANES)]
        u = _monotonic_u32(xv)
        m = u >= thresh
        iv = lane + i * _LANES
        cs = plsc.cumsum(m.astype(jnp.int32))
        plsc.store_scatter(cidx_v, [cntv + cs - 1], iv, mask=m)
        return cntv + _perm(cs, lane15)
      return cntv[0]

    @plsc.parallel_loop(0, nchunks, unroll=8, carry=zeros16 - 1)
    def cntm(i, cntm):
      xv = x_v[pl.ds(base + i * _LANES, _LANES)]
      m = xv >= 2.7
      iv = lane + i * _LANES
      cs = plsc.cumsum(m.astype(jnp.int32))
      plsc.store_scatter(cidx_v, [cntm + cs], iv, mask=m)
      return cntm + _perm(cs, lane15)
    cnt0 = cntm[0] + 1

    def fallback():
      @plsc.parallel_loop(0, _NBINS // _LANES, unroll=8)
      def _(i):
        hist_v[pl.ds(i * _LANES, _LANES)] = zeros16

      ones = jnp.ones((_LANES,), jnp.int32)

      @plsc.parallel_loop(0, nchunks, unroll=8)
      def _(i):
        xv = x_v[pl.ds(base + i * _LANES, _LANES)]
        u = _monotonic_u32(xv)
        d = (u >> _BIN_SHIFT).astype(jnp.int32)
        plsc.addupdate_scatter(hist_v, [d], ones, mask=lane >= 0)

      def scond(c):
        b, cum = c
        return (cum < k) & (b >= 0)

      def sbody(c):
        b, cum = c
        hv = hist_v[pl.ds(b * _LANES, _LANES)]
        return b - 1, cum + jnp.sum(hv)
      cfin, cum = lax.while_loop(
          scond, sbody, (jnp.int32(_NBINS // _LANES - 1), jnp.int32(0)))
      cc = jnp.maximum(cfin + 1, 0)
      hv = hist_v[pl.ds(cc * _LANES, _LANES)]
      cum_above = cum - jnp.sum(hv)
      cs = jnp.cumsum(lax.rev(hv, (0,)))
      p = jnp.sum((cum_above + cs < k).astype(jnp.int32))
      binsel = jnp.where(cum >= k, cc * _LANES + (_LANES - 1) - p, 0)
      thresh = binsel.astype(jnp.uint32) << _BIN_SHIFT

      @plsc.parallel_loop(0, _NV, unroll=4)
      def _(j):
        cidx_v[pl.ds(j * _LANES, _LANES)] = lane + jnp.int32(n)

      tiered_select(compact(thresh))
    def make_fast(nv):
      def fast_sel():
        ivs = [cidx_v[pl.ds(j * _LANES, _LANES)] for j in range(nv)]
        vvs0 = tuple(plsc.load_gather(x_v, [iv + base16]) for iv in ivs)

        def step(j, vvs):
          red = [(vvs[t], ivs[t]) for t in range(nv)]
          while len(red) > 1:
            half = len(red) // 2
            red = [_combine(*red[t], *red[t + half]) for t in range(half)]
          bv, bi = _argmax_splat(*red[0])
          jsplat = zeros16 + j
          plsc.store_scatter(oval_v, [rsplat, jsplat], bv, mask=lane0)
          plsc.store_scatter(oidx_v, [rsplat, jsplat], bi, mask=lane0)
          return tuple(jnp.where(ivs[t] == bi, neg_inf, vvs[t])
                       for t in range(nv))
        lax.fori_loop(0, k, step, vvs0)
      return fast_sel

    def make_slow(cnt):
     def slow_sel():
      cidx_v[pl.ds(cnt, _LANES)] = lane + jnp.int32(n)
      nvec = (cnt + _LANES - 1) // _LANES
      init_sel = (ninf16, jnp.full((_LANES,), big, jnp.int32))

      def sel(j, _):
        @plsc.parallel_loop(0, nvec, unroll=2, carry=init_sel)
        def sweep(v, carry):
          bv, bi = carry
          iv = cidx_v[pl.ds(v * _LANES, _LANES)]
          vv = plsc.load_gather(x_v, [iv + base16])
          return _combine(bv, bi, vv, iv)
        bv, bi = _argmax_splat(*sweep)
        jsplat = zeros16 + j
        plsc.store_scatter(oval_v, [rsplat, jsplat], bv, mask=lane0)
        plsc.store_scatter(oidx_v, [rsplat, jsplat], bi, mask=lane0)
        plsc.store_scatter(x_v, [bi + base16], ninf16, mask=lane0)
        return 0
      lax.fori_loop(0, k, sel, 0)
     return slow_sel

    def tiered_select(cnt):
      lax.cond(cnt <= 8 * _LANES, make_fast(8),
               lambda: lax.cond(cnt <= _NV * _LANES, make_fast(_NV),
                                make_slow(cnt)))

    lax.cond((cnt0 >= k) & (cnt0 <= _NV * _LANES),
             lambda: tiered_select(cnt0), fallback)
    return 0

  lax.fori_loop(0, rows_per_w, do_row, 0)

  c0 = pltpu.make_async_copy(
      oidx_v, out_idx_hbm.at[pl.ds(wid * rows_per_w, rows_per_w)],
      osem_v.at[0])
  c1 = pltpu.make_async_copy(
      oval_v, out_val_hbm.at[pl.ds(wid * rows_per_w, rows_per_w)],
      osem_v.at[1])
  c0.start()
  c1.start()
  c0.wait()
  c1.wait()


def kernel(scores):
  b, n = scores.shape
  k = min(64, n)
  assert b % _NW == 0 and n % _LANES == 0
  rows_per_w = b // _NW
  body = functools.partial(_topk_body, n, k, rows_per_w)
  f = pl.kernel(
      body,
      out_type=(jax.ShapeDtypeStruct((b, k), jnp.int32),
                jax.ShapeDtypeStruct((b, k), jnp.float32)),
      mesh=plsc.VectorSubcoreMesh(core_axis_name="c", subcore_axis_name="s",
                                  num_cores=_NCORES, num_subcores=_NSUB),
      compiler_params=pltpu.CompilerParams(needs_layout_passes=False),
      scratch_types=[
          pltpu.VMEM((2 * (n + 128),), jnp.float32),
          pltpu.VMEM((n + _LANES,), jnp.int32),
          pltpu.VMEM((_NBINS,), jnp.int32),
          pltpu.VMEM((b // _NW, 64), jnp.float32),
          pltpu.VMEM((b // _NW, 64), jnp.int32),
          pltpu.SemaphoreType.DMA((2,)),
          pltpu.SemaphoreType.DMA((2,)),
      ],
  )
  return f(scores)
